# baseline (speedup 1.0000x reference)
import functools

import jax
import jax.numpy as jnp
from jax import lax
from jax.experimental import pallas as pl
from jax.experimental.pallas import tpu as pltpu
from jax.experimental.pallas import tpu_sc as plsc

F = 64
CAV = 5
NY = 504
NX = 504
NP = NY * NX
TOTAL = CAV * F * NP
NPIL = 40000
NPAD = 40960
CHUNK = 2048
NCHUNK = NPAD // CHUNK
ZWORDS = NP // 4


def _sc_scatter_kernel(b_hbm, y_hbm, x_hbm, feat_hbm, out_hbm,
                       zero_v, b_v, y_v, x_v, base_v, vals_v, dest_v,
                       semz, sem):
    info = plsc.get_sparse_core_info()
    nc = info.num_cores
    wid = lax.axis_index("s") * nc + lax.axis_index("c")
    f0 = wid * 2

    zeros16 = jnp.zeros((16,), jnp.float32)

    def mset(i, carry):
        zero_v[pl.ds(i * 16, 16)] = zeros16
        return carry

    lax.fori_loop(0, ZWORDS // 16, mset, 0)

    for b in range(CAV):
        off = (b * F + f0) * NP
        copies = [
            pltpu.async_copy(zero_v, out_hbm.at[pl.ds(off + k * ZWORDS, ZWORDS)],
                             semz)
            for k in range(8)
        ]
        for cp in copies:
            cp.wait()

    def chunk_body(ci, carry):
        c0 = ci * CHUNK
        pltpu.sync_copy(b_hbm.at[pl.ds(c0, CHUNK)], b_v)
        pltpu.sync_copy(y_hbm.at[pl.ds(c0, CHUNK)], y_v)
        pltpu.sync_copy(x_hbm.at[pl.ds(c0, CHUNK)], x_v)
        for i in range(CHUNK // 16):
            sl = pl.ds(i * 16, 16)
            base_v[sl] = b_v[sl] * (F * NP) + y_v[sl] * NX + x_v[sl]
        for fi in range(2):
            f = f0 + fi
            pltpu.sync_copy(feat_hbm.at[f, pl.ds(c0, CHUNK)], vals_v)
            foff = f * NP
            for i in range(CHUNK // 16):
                dest_v[i // 8, pl.ds((i % 8) * 16, 16)] = (
                    base_v[pl.ds(i * 16, 16)] + foff)
            copies = [
                pltpu.async_copy(vals_v.at[pl.ds(j * 128, 128)],
                                 out_hbm.at[dest_v.at[j]], sem)
                for j in range(16)
            ]
            for cp in copies:
                cp.wait()
        return carry

    lax.fori_loop(0, NCHUNK, chunk_body, 0)


@jax.jit
def _run(b_col, y_col, x_col, feat_t):
    mesh = plsc.VectorSubcoreMesh(core_axis_name="c", subcore_axis_name="s")
    k = functools.partial(
        pl.kernel,
        mesh=mesh,
        out_type=jax.ShapeDtypeStruct((TOTAL,), jnp.float32),
        scratch_types=[
            pltpu.VMEM((ZWORDS,), jnp.float32),
            pltpu.VMEM((CHUNK,), jnp.int32),
            pltpu.VMEM((CHUNK,), jnp.int32),
            pltpu.VMEM((CHUNK,), jnp.int32),
            pltpu.VMEM((CHUNK,), jnp.int32),
            pltpu.VMEM((CHUNK,), jnp.float32),
            pltpu.VMEM((16, 128), jnp.int32),
            pltpu.SemaphoreType.DMA,
            pltpu.SemaphoreType.DMA,
        ],
    )(_sc_scatter_kernel)
    flat = k(b_col, y_col, x_col, feat_t)
    return flat.reshape(CAV, F, NY, NX)


def kernel(voxel_coords, pillar_features):
    npad = NPAD - NPIL
    b_col = jnp.concatenate(
        [voxel_coords[:, 0], jnp.broadcast_to(voxel_coords[0, 0], (npad,))])
    y_col = jnp.concatenate(
        [voxel_coords[:, 2], jnp.broadcast_to(voxel_coords[0, 2], (npad,))])
    x_col = jnp.concatenate(
        [voxel_coords[:, 3], jnp.broadcast_to(voxel_coords[0, 3], (npad,))])
    feat_pad = jnp.concatenate(
        [pillar_features,
         jnp.broadcast_to(pillar_features[0], (npad, F))], axis=0)
    feat_t = feat_pad.T
    return _run(b_col.astype(jnp.int32), y_col.astype(jnp.int32),
                x_col.astype(jnp.int32), feat_t)
